# baseline (speedup 1.0000x reference)
import jax
import jax.numpy as jnp
from jax.experimental import pallas as pl
from jax.experimental.pallas import tpu as pltpu

B, S, D = 4, 2048, 4096
E, K = 8, 2

S_CHUNK = 256
CPS = S // S_CHUNK
NC = B * CPS
D_CHUNK = 256
NJ = D // D_CHUNK
N_RES = 13
SPLIT = 2 * N_RES

T_POOL = NC
T_MM = T_POOL + NJ
T_END = T_MM + NC


def _routing_tail(h, w2, b2, pen, gamma, beta):
    logits = jnp.dot(h, w2, preferred_element_type=jnp.float32) + b2[None, :]
    m = jnp.max(logits, axis=-1, keepdims=True)
    ex = jnp.exp(logits - m)
    route_prob = ex / jnp.sum(ex, axis=-1, keepdims=True)
    biased = route_prob - pen[None, :]

    eidx = jax.lax.broadcasted_iota(jnp.int32, (B, E), 1)
    big = jnp.int32(E)
    m1 = jnp.max(biased, axis=-1, keepdims=True)
    i1 = jnp.min(jnp.where(biased == m1, eidx, big), axis=-1, keepdims=True)
    masked = jnp.where(eidx == i1, -jnp.inf, biased)
    m2 = jnp.max(masked, axis=-1, keepdims=True)
    i2 = jnp.min(jnp.where(masked == m2, eidx, big), axis=-1, keepdims=True)

    denom = m1 + m2
    is1 = (eidx == i1)
    is2 = (eidx == i2)
    coeff = jnp.where(is1, m1 / denom, 0.0) + jnp.where(is2, m2 / denom, 0.0)

    cnt = jnp.sum(is1.astype(jnp.float32) + is2.astype(jnp.float32),
                  axis=0, keepdims=True)
    importance = jnp.mean(route_prob, axis=0, keepdims=True)
    load = cnt / jnp.maximum(jnp.sum(cnt), 1.0)
    lb = jnp.float32(E) * jnp.sum(importance * load)

    g = jnp.dot(coeff, gamma, preferred_element_type=jnp.float32)
    bvec = jnp.dot(coeff, beta, preferred_element_type=jnp.float32)
    return coeff, lb, g, bvec


def _norm_block(xb, g, bvec):
    s1 = jnp.mean(xb, axis=-1, keepdims=True)
    s2 = jnp.mean(xb * xb, axis=-1, keepdims=True)
    var = s2 - s1 * s1
    rstd = jax.lax.rsqrt(var + 1e-6)
    return (xb - s1) * rstd * g + bvec


def _mega_kernel(x_ref, w1_ref, b1_ref, w2_ref, b2_ref, pen_ref,
                 gamma_ref, beta_ref,
                 o_ref, coeff_ref, lb_ref,
                 acc_pool, acc_h, gb_scr, res):
    t = pl.program_id(0)

    @pl.when(t == 0)
    def _():
        acc_pool[...] = jnp.zeros_like(acc_pool)

    @pl.when(t < T_POOL)
    def _():
        bidx = t // CPS
        xb = x_ref[0]
        acc_pool[bidx, :] += jnp.sum(xb, axis=0) * (1.0 / S)

        @pl.when(t < N_RES)
        def _():
            res[t] = xb.astype(jnp.bfloat16)

    @pl.when(t == T_POOL)
    def _():
        acc_h[...] = jnp.zeros_like(acc_h)

    @pl.when((t >= T_POOL) & (t < T_MM))
    def _():
        j = t - T_POOL
        acc_h[...] += jnp.dot(acc_pool[:, pl.ds(j * D_CHUNK, D_CHUNK)],
                              w1_ref[...], preferred_element_type=jnp.float32)

    @pl.when(t == T_MM - 1)
    def _():
        h = jnp.maximum(acc_h[...] + b1_ref[...][None, :], 0.0)
        coeff, lb, g, bvec = _routing_tail(
            h, w2_ref[...], b2_ref[...], pen_ref[...],
            gamma_ref[...], beta_ref[...])
        coeff_ref[...] = coeff
        lb_ref[...] = lb.reshape(1, 1)
        gb_scr[0] = g
        gb_scr[1] = bvec

    @pl.when(t >= T_MM)
    def _():
        c = t - T_MM
        chunk = _perm(c)
        bidx = chunk // CPS
        g = gb_scr[0, bidx][None, :]
        bvec = gb_scr[1, bidx][None, :]
        is_res = (c < SPLIT) & (c % 2 == 0)

        @pl.when(is_res)
        def _():
            o_ref[0] = _norm_block(res[c // 2].astype(jnp.float32), g, bvec)

        @pl.when(jnp.logical_not(is_res))
        def _():
            o_ref[0] = _norm_block(x_ref[0], g, bvec)


def _perm(c):
    return jnp.where(c >= SPLIT, c,
                     jnp.where(c % 2 == 0, c // 2, N_RES + c // 2))


def _xstream(c):
    return jnp.where(c >= SPLIT, c, N_RES + c // 2)


def _x_idx(t):
    c = jnp.where(t < T_POOL, t,
                  jnp.where(t < T_MM, NC - 1, _xstream(t - T_MM)))
    return (c // CPS, c % CPS, 0)


def _o_idx(t):
    c = jnp.where(t < T_MM, 0, _perm(t - T_MM))
    return (c // CPS, c % CPS, 0)


def _w1_idx(t):
    j = jnp.clip(t - T_POOL, 0, NJ - 1)
    return (j, 0)


@jax.jit
def kernel(x, router_w1, router_b1, router_w2, router_b2, gamma, beta, penalty):
    out, coeff, lb = pl.pallas_call(
        _mega_kernel,
        grid=(T_END,),
        in_specs=[
            pl.BlockSpec((1, S_CHUNK, D), _x_idx),
            pl.BlockSpec((D_CHUNK, D), _w1_idx),
            pl.BlockSpec((D,), lambda t: (0,)),
            pl.BlockSpec((D, E), lambda t: (0, 0)),
            pl.BlockSpec((E,), lambda t: (0,)),
            pl.BlockSpec((E,), lambda t: (0,)),
            pl.BlockSpec((E, D), lambda t: (0, 0)),
            pl.BlockSpec((E, D), lambda t: (0, 0)),
        ],
        out_specs=[
            pl.BlockSpec((1, S_CHUNK, D), _o_idx),
            pl.BlockSpec((B, E), lambda t: (0, 0)),
            pl.BlockSpec((1, 1), lambda t: (0, 0)),
        ],
        out_shape=[
            jax.ShapeDtypeStruct((B, S, D), jnp.float32),
            jax.ShapeDtypeStruct((B, E), jnp.float32),
            jax.ShapeDtypeStruct((1, 1), jnp.float32),
        ],
        scratch_shapes=[
            pltpu.VMEM((B, D), jnp.float32),
            pltpu.VMEM((B, D), jnp.float32),
            pltpu.VMEM((2, B, D), jnp.float32),
            pltpu.VMEM((N_RES, S_CHUNK, D), jnp.bfloat16),
        ],
        compiler_params=pltpu.CompilerParams(
            dimension_semantics=("arbitrary",)),
    )(x, router_w1, router_b1, router_w2, router_b2, penalty, gamma, beta)

    return (out, coeff, lb.reshape(()))


# speedup vs baseline: 1.1255x; 1.1255x over previous
import jax
import jax.numpy as jnp
from jax.experimental import pallas as pl
from jax.experimental.pallas import tpu as pltpu

B, S, D = 4, 2048, 4096
E, K = 8, 2

S_CHUNK = 256
CPS = S // S_CHUNK
NC = B * CPS
D_CHUNK = 256
NJ = D // D_CHUNK
N_RES = 13

T_POOL = NC
T_MM = T_POOL + NJ
T_END = T_MM + NC


def _routing_tail(h, w2, b2, pen, gamma, beta):
    logits = jnp.dot(h, w2, preferred_element_type=jnp.float32) + b2[None, :]
    m = jnp.max(logits, axis=-1, keepdims=True)
    ex = jnp.exp(logits - m)
    route_prob = ex / jnp.sum(ex, axis=-1, keepdims=True)
    biased = route_prob - pen[None, :]

    eidx = jax.lax.broadcasted_iota(jnp.int32, (B, E), 1)
    big = jnp.int32(E)
    m1 = jnp.max(biased, axis=-1, keepdims=True)
    i1 = jnp.min(jnp.where(biased == m1, eidx, big), axis=-1, keepdims=True)
    masked = jnp.where(eidx == i1, -jnp.inf, biased)
    m2 = jnp.max(masked, axis=-1, keepdims=True)
    i2 = jnp.min(jnp.where(masked == m2, eidx, big), axis=-1, keepdims=True)

    denom = m1 + m2
    is1 = (eidx == i1)
    is2 = (eidx == i2)
    coeff = jnp.where(is1, m1 / denom, 0.0) + jnp.where(is2, m2 / denom, 0.0)

    cnt = jnp.sum(is1.astype(jnp.float32) + is2.astype(jnp.float32),
                  axis=0, keepdims=True)
    importance = jnp.mean(route_prob, axis=0, keepdims=True)
    load = cnt / jnp.maximum(jnp.sum(cnt), 1.0)
    lb = jnp.float32(E) * jnp.sum(importance * load)

    g = jnp.dot(coeff, gamma, preferred_element_type=jnp.float32)
    bvec = jnp.dot(coeff, beta, preferred_element_type=jnp.float32)
    return coeff, lb, g, bvec


def _norm_block(xb, g, bvec):
    s1 = jnp.mean(xb, axis=-1, keepdims=True)
    s2 = jnp.mean(xb * xb, axis=-1, keepdims=True)
    rstd = jax.lax.rsqrt(s2 - s1 * s1 + 1e-6)
    return (xb - s1) * rstd * g + bvec


def _mega_kernel(x_ref, w1_ref, b1_ref, w2_ref, b2_ref, pen_ref,
                 gamma_ref, beta_ref,
                 o_ref, coeff_ref, lb_ref,
                 acc_pool, acc_h, gb_scr, res):
    t = pl.program_id(0)

    @pl.when(t == 0)
    def _():
        acc_pool[...] = jnp.zeros_like(acc_pool)

    @pl.when(t < T_POOL)
    def _():
        bidx = t // CPS
        xb = x_ref[0]
        acc_pool[bidx, :] += jnp.sum(xb, axis=0) * (1.0 / S)

        @pl.when(t < N_RES)
        def _():
            res[t] = xb.astype(jnp.bfloat16)

    @pl.when(t == T_POOL)
    def _():
        acc_h[...] = jnp.zeros_like(acc_h)

    @pl.when((t >= T_POOL) & (t < T_MM))
    def _():
        j = t - T_POOL
        acc_h[...] += jnp.dot(acc_pool[:, pl.ds(j * D_CHUNK, D_CHUNK)],
                              w1_ref[...], preferred_element_type=jnp.float32)

    @pl.when(t == T_MM - 1)
    def _():
        h = jnp.maximum(acc_h[...] + b1_ref[...][None, :], 0.0)
        coeff, lb, g, bvec = _routing_tail(
            h, w2_ref[...], b2_ref[...], pen_ref[...],
            gamma_ref[...], beta_ref[...])
        coeff_ref[...] = coeff
        lb_ref[...] = lb.reshape(1, 1)
        gb_scr[0] = g
        gb_scr[1] = bvec

    @pl.when(t >= T_MM)
    def _():
        c = t - T_MM
        bidx = c // CPS
        g = gb_scr[0, bidx][None, :]
        bvec = gb_scr[1, bidx][None, :]

        @pl.when(c < N_RES)
        def _():
            o_ref[0] = _norm_block(res[c].astype(jnp.float32), g, bvec)

        @pl.when(c >= N_RES)
        def _():
            o_ref[0] = _norm_block(x_ref[0], g, bvec)


def _x_idx(t):
    c = jnp.where(t < T_POOL, t, jnp.where(t < T_MM + N_RES, NC - 1, t - T_MM))
    return (c // CPS, c % CPS, 0)


def _o_idx(t):
    c = jnp.where(t < T_MM, 0, t - T_MM)
    return (c // CPS, c % CPS, 0)


def _w1_idx(t):
    j = jnp.clip(t - T_POOL, 0, NJ - 1)
    return (j, 0)


@jax.jit
def kernel(x, router_w1, router_b1, router_w2, router_b2, gamma, beta, penalty):
    out, coeff, lb = pl.pallas_call(
        _mega_kernel,
        grid=(T_END,),
        in_specs=[
            pl.BlockSpec((1, S_CHUNK, D), _x_idx),
            pl.BlockSpec((D_CHUNK, D), _w1_idx),
            pl.BlockSpec((D,), lambda t: (0,)),
            pl.BlockSpec((D, E), lambda t: (0, 0)),
            pl.BlockSpec((E,), lambda t: (0,)),
            pl.BlockSpec((E,), lambda t: (0,)),
            pl.BlockSpec((E, D), lambda t: (0, 0)),
            pl.BlockSpec((E, D), lambda t: (0, 0)),
        ],
        out_specs=[
            pl.BlockSpec((1, S_CHUNK, D), _o_idx),
            pl.BlockSpec((B, E), lambda t: (0, 0)),
            pl.BlockSpec((1, 1), lambda t: (0, 0)),
        ],
        out_shape=[
            jax.ShapeDtypeStruct((B, S, D), jnp.float32),
            jax.ShapeDtypeStruct((B, E), jnp.float32),
            jax.ShapeDtypeStruct((1, 1), jnp.float32),
        ],
        scratch_shapes=[
            pltpu.VMEM((B, D), jnp.float32),
            pltpu.VMEM((B, D), jnp.float32),
            pltpu.VMEM((2, B, D), jnp.float32),
            pltpu.VMEM((N_RES, S_CHUNK, D), jnp.bfloat16),
        ],
        compiler_params=pltpu.CompilerParams(
            dimension_semantics=("arbitrary",)),
    )(x, router_w1, router_b1, router_w2, router_b2, penalty, gamma, beta)

    return (out, coeff, lb.reshape(()))


# speedup vs baseline: 1.1420x; 1.0147x over previous
import jax
import jax.numpy as jnp
from jax.experimental import pallas as pl
from jax.experimental.pallas import tpu as pltpu

B, S, D = 4, 2048, 4096
E, K = 8, 2

S_CHUNK = 512
CPS = S // S_CHUNK
NC = B * CPS
D_CHUNK = 256
NJ = D // D_CHUNK
N_RES = 1

T_POOL = NC
T_MM = T_POOL + NJ
T_END = T_MM + NC


def _routing_tail(h, w2, b2, pen, gamma, beta):
    logits = jnp.dot(h, w2, preferred_element_type=jnp.float32) + b2[None, :]
    m = jnp.max(logits, axis=-1, keepdims=True)
    ex = jnp.exp(logits - m)
    route_prob = ex / jnp.sum(ex, axis=-1, keepdims=True)
    biased = route_prob - pen[None, :]

    eidx = jax.lax.broadcasted_iota(jnp.int32, (B, E), 1)
    big = jnp.int32(E)
    m1 = jnp.max(biased, axis=-1, keepdims=True)
    i1 = jnp.min(jnp.where(biased == m1, eidx, big), axis=-1, keepdims=True)
    masked = jnp.where(eidx == i1, -jnp.inf, biased)
    m2 = jnp.max(masked, axis=-1, keepdims=True)
    i2 = jnp.min(jnp.where(masked == m2, eidx, big), axis=-1, keepdims=True)

    denom = m1 + m2
    is1 = (eidx == i1)
    is2 = (eidx == i2)
    coeff = jnp.where(is1, m1 / denom, 0.0) + jnp.where(is2, m2 / denom, 0.0)

    cnt = jnp.sum(is1.astype(jnp.float32) + is2.astype(jnp.float32),
                  axis=0, keepdims=True)
    importance = jnp.mean(route_prob, axis=0, keepdims=True)
    load = cnt / jnp.maximum(jnp.sum(cnt), 1.0)
    lb = jnp.float32(E) * jnp.sum(importance * load)

    g = jnp.dot(coeff, gamma, preferred_element_type=jnp.float32)
    bvec = jnp.dot(coeff, beta, preferred_element_type=jnp.float32)
    return coeff, lb, g, bvec


def _norm_block(xb, g, bvec):
    s1 = jnp.mean(xb, axis=-1, keepdims=True)
    s2 = jnp.mean(xb * xb, axis=-1, keepdims=True)
    rstd = jax.lax.rsqrt(s2 - s1 * s1 + 1e-6)
    return (xb - s1) * rstd * g + bvec


def _mega_kernel(x_ref, w1_ref, b1_ref, w2_ref, b2_ref, pen_ref,
                 gamma_ref, beta_ref,
                 o_ref, coeff_ref, lb_ref,
                 acc_pool, acc_h, gb_scr, res):
    t = pl.program_id(0)

    @pl.when(t == 0)
    def _():
        acc_pool[...] = jnp.zeros_like(acc_pool)

    @pl.when(t < T_POOL)
    def _():
        bidx = t // CPS
        xb = x_ref[0]
        acc_pool[bidx, :] += jnp.sum(xb, axis=0) * (1.0 / S)

        @pl.when(t < N_RES)
        def _():
            res[t] = xb.astype(jnp.bfloat16)

    @pl.when(t == T_POOL)
    def _():
        acc_h[...] = jnp.zeros_like(acc_h)

    @pl.when((t >= T_POOL) & (t < T_MM))
    def _():
        j = t - T_POOL
        acc_h[...] += jnp.dot(acc_pool[:, pl.ds(j * D_CHUNK, D_CHUNK)],
                              w1_ref[...], preferred_element_type=jnp.float32)

    @pl.when(t == T_MM - 1)
    def _():
        h = jnp.maximum(acc_h[...] + b1_ref[...][None, :], 0.0)
        coeff, lb, g, bvec = _routing_tail(
            h, w2_ref[...], b2_ref[...], pen_ref[...],
            gamma_ref[...], beta_ref[...])
        coeff_ref[...] = coeff
        lb_ref[...] = lb.reshape(1, 1)
        gb_scr[0] = g
        gb_scr[1] = bvec

    @pl.when(t >= T_MM)
    def _():
        c = t - T_MM
        bidx = c // CPS
        g = gb_scr[0, bidx][None, :]
        bvec = gb_scr[1, bidx][None, :]

        @pl.when(c < N_RES)
        def _():
            o_ref[0] = _norm_block(res[c].astype(jnp.float32), g, bvec)

        @pl.when(c >= N_RES)
        def _():
            o_ref[0] = _norm_block(x_ref[0], g, bvec)


def _x_idx(t):
    c = jnp.where(t < T_POOL, t, jnp.where(t < T_MM + N_RES, NC - 1, t - T_MM))
    return (c // CPS, c % CPS, 0)


def _o_idx(t):
    c = jnp.where(t < T_MM, 0, t - T_MM)
    return (c // CPS, c % CPS, 0)


def _w1_idx(t):
    j = jnp.clip(t - T_POOL, 0, NJ - 1)
    return (j, 0)


@jax.jit
def kernel(x, router_w1, router_b1, router_w2, router_b2, gamma, beta, penalty):
    out, coeff, lb = pl.pallas_call(
        _mega_kernel,
        grid=(T_END,),
        in_specs=[
            pl.BlockSpec((1, S_CHUNK, D), _x_idx),
            pl.BlockSpec((D_CHUNK, D), _w1_idx),
            pl.BlockSpec((D,), lambda t: (0,)),
            pl.BlockSpec((D, E), lambda t: (0, 0)),
            pl.BlockSpec((E,), lambda t: (0,)),
            pl.BlockSpec((E,), lambda t: (0,)),
            pl.BlockSpec((E, D), lambda t: (0, 0)),
            pl.BlockSpec((E, D), lambda t: (0, 0)),
        ],
        out_specs=[
            pl.BlockSpec((1, S_CHUNK, D), _o_idx),
            pl.BlockSpec((B, E), lambda t: (0, 0)),
            pl.BlockSpec((1, 1), lambda t: (0, 0)),
        ],
        out_shape=[
            jax.ShapeDtypeStruct((B, S, D), jnp.float32),
            jax.ShapeDtypeStruct((B, E), jnp.float32),
            jax.ShapeDtypeStruct((1, 1), jnp.float32),
        ],
        scratch_shapes=[
            pltpu.VMEM((B, D), jnp.float32),
            pltpu.VMEM((B, D), jnp.float32),
            pltpu.VMEM((2, B, D), jnp.float32),
            pltpu.VMEM((N_RES, S_CHUNK, D), jnp.bfloat16),
        ],
        compiler_params=pltpu.CompilerParams(
            dimension_semantics=("arbitrary",)),
    )(x, router_w1, router_b1, router_w2, router_b2, penalty, gamma, beta)

    return (out, coeff, lb.reshape(()))


# speedup vs baseline: 1.1786x; 1.0321x over previous
import jax
import jax.numpy as jnp
from jax.experimental import pallas as pl
from jax.experimental.pallas import tpu as pltpu

B, S, D = 4, 2048, 4096
E, K = 8, 2

S_CHUNK = 512
CPS = S // S_CHUNK
NC = B * CPS
D_CHUNK = 256
NJ = D // D_CHUNK
N_RES = 3

T_POOL = NC
T_MM = T_POOL + NJ
T_END = T_MM + NC


def _routing_tail(h, w2t, b2, pen, gamma, beta):
    logits = jax.lax.dot_general(
        h, w2t, (((1,), (1,)), ((), ())),
        preferred_element_type=jnp.float32) + b2[None, :]
    m = jnp.max(logits, axis=-1, keepdims=True)
    ex = jnp.exp(logits - m)
    route_prob = ex / jnp.sum(ex, axis=-1, keepdims=True)
    biased = route_prob - pen[None, :]

    eidx = jax.lax.broadcasted_iota(jnp.int32, (B, E), 1)
    big = jnp.int32(E)
    m1 = jnp.max(biased, axis=-1, keepdims=True)
    i1 = jnp.min(jnp.where(biased == m1, eidx, big), axis=-1, keepdims=True)
    masked = jnp.where(eidx == i1, -jnp.inf, biased)
    m2 = jnp.max(masked, axis=-1, keepdims=True)
    i2 = jnp.min(jnp.where(masked == m2, eidx, big), axis=-1, keepdims=True)

    denom = m1 + m2
    is1 = (eidx == i1)
    is2 = (eidx == i2)
    coeff = jnp.where(is1, m1 / denom, 0.0) + jnp.where(is2, m2 / denom, 0.0)

    cnt = jnp.sum(is1.astype(jnp.float32) + is2.astype(jnp.float32),
                  axis=0, keepdims=True)
    importance = jnp.mean(route_prob, axis=0, keepdims=True)
    load = cnt / jnp.maximum(jnp.sum(cnt), 1.0)
    lb = jnp.float32(E) * jnp.sum(importance * load)

    g = jnp.dot(coeff, gamma, preferred_element_type=jnp.float32)
    bvec = jnp.dot(coeff, beta, preferred_element_type=jnp.float32)
    return coeff, lb, g, bvec


def _norm_block(xb, g, bvec):
    s1 = jnp.mean(xb, axis=-1, keepdims=True)
    s2 = jnp.mean(xb * xb, axis=-1, keepdims=True)
    rstd = jax.lax.rsqrt(s2 - s1 * s1 + 1e-6)
    return (xb - s1) * rstd * g + bvec


def _mega_kernel(x_ref, w1_ref, b1_ref, w2_ref, b2_ref, pen_ref,
                 gamma_ref, beta_ref,
                 o_ref, coeff_ref, lb_ref,
                 acc_pool, acc_h, gb_scr, res):
    t = pl.program_id(0)

    @pl.when(t == 0)
    def _():
        acc_pool[...] = jnp.zeros_like(acc_pool)

    @pl.when(t < T_POOL)
    def _():
        bidx = t // CPS
        xb = x_ref[0]
        acc_pool[bidx, :] += jnp.sum(xb, axis=0) * (1.0 / S)

        @pl.when(t < N_RES)
        def _():
            res[t] = xb.astype(jnp.bfloat16)

    @pl.when(t == T_POOL)
    def _():
        acc_h[...] = jnp.zeros_like(acc_h)

    @pl.when((t >= T_POOL) & (t < T_MM))
    def _():
        j = t - T_POOL
        acc_h[...] += jnp.dot(acc_pool[:, pl.ds(j * D_CHUNK, D_CHUNK)],
                              w1_ref[...], preferred_element_type=jnp.float32)

    @pl.when(t == T_MM - 1)
    def _():
        h = jnp.maximum(acc_h[...] + b1_ref[...][None, :], 0.0)
        coeff, lb, g, bvec = _routing_tail(
            h, w2_ref[...], b2_ref[...], pen_ref[...],
            gamma_ref[...], beta_ref[...])
        coeff_ref[...] = coeff
        lb_ref[...] = lb.reshape(1, 1)
        gb_scr[0] = g
        gb_scr[1] = bvec

    @pl.when(t >= T_MM)
    def _():
        c = t - T_MM
        bidx = c // CPS
        g = gb_scr[0, bidx][None, :]
        bvec = gb_scr[1, bidx][None, :]

        @pl.when(c < N_RES)
        def _():
            o_ref[0] = _norm_block(res[c].astype(jnp.float32), g, bvec)

        @pl.when(c >= N_RES)
        def _():
            o_ref[0] = _norm_block(x_ref[0], g, bvec)


def _x_idx(t):
    c = jnp.where(t < T_POOL, t, jnp.where(t < T_MM + N_RES, NC - 1, t - T_MM))
    return (c // CPS, c % CPS, 0)


def _o_idx(t):
    c = jnp.where(t < T_MM, 0, t - T_MM)
    return (c // CPS, c % CPS, 0)


def _w1_idx(t):
    j = jnp.clip(t - T_POOL, 0, NJ - 1)
    return (j, 0)


@jax.jit
def kernel(x, router_w1, router_b1, router_w2, router_b2, gamma, beta, penalty):
    out, coeff, lb = pl.pallas_call(
        _mega_kernel,
        grid=(T_END,),
        in_specs=[
            pl.BlockSpec((1, S_CHUNK, D), _x_idx),
            pl.BlockSpec((D_CHUNK, D), _w1_idx),
            pl.BlockSpec((D,), lambda t: (0,)),
            pl.BlockSpec((E, D), lambda t: (0, 0)),
            pl.BlockSpec((E,), lambda t: (0,)),
            pl.BlockSpec((E,), lambda t: (0,)),
            pl.BlockSpec((E, D), lambda t: (0, 0)),
            pl.BlockSpec((E, D), lambda t: (0, 0)),
        ],
        out_specs=[
            pl.BlockSpec((1, S_CHUNK, D), _o_idx),
            pl.BlockSpec((B, E), lambda t: (0, 0)),
            pl.BlockSpec((1, 1), lambda t: (0, 0)),
        ],
        out_shape=[
            jax.ShapeDtypeStruct((B, S, D), jnp.float32),
            jax.ShapeDtypeStruct((B, E), jnp.float32),
            jax.ShapeDtypeStruct((1, 1), jnp.float32),
        ],
        scratch_shapes=[
            pltpu.VMEM((B, D), jnp.float32),
            pltpu.VMEM((B, D), jnp.float32),
            pltpu.VMEM((2, B, D), jnp.float32),
            pltpu.VMEM((N_RES, S_CHUNK, D), jnp.bfloat16),
        ],
        compiler_params=pltpu.CompilerParams(
            dimension_semantics=("arbitrary",),
            vmem_limit_bytes=67108864),
    )(x, router_w1, router_b1, router_w2.T, router_b2, penalty, gamma, beta)

    return (out, coeff, lb.reshape(()))


# speedup vs baseline: 1.1877x; 1.0077x over previous
import jax
import jax.numpy as jnp
from jax.experimental import pallas as pl
from jax.experimental.pallas import tpu as pltpu

B, S, D = 4, 2048, 4096
E, K = 8, 2

S_CHUNK = 512
CPS = S // S_CHUNK
NC = B * CPS
D_CHUNK = 256
NJ = D // D_CHUNK
N_RES = 3

T_POOL = NC
T_MM = T_POOL + NJ
T_END = T_MM + NC


def _routing_tail(h, w2t, b2, pen, gamma, beta):
    logits = jax.lax.dot_general(
        h, w2t, (((1,), (1,)), ((), ())),
        preferred_element_type=jnp.float32) + b2[None, :]
    m = jnp.max(logits, axis=-1, keepdims=True)
    ex = jnp.exp(logits - m)
    route_prob = ex / jnp.sum(ex, axis=-1, keepdims=True)
    biased = route_prob - pen[None, :]

    eidx = jax.lax.broadcasted_iota(jnp.int32, (B, E), 1)
    big = jnp.int32(E)
    m1 = jnp.max(biased, axis=-1, keepdims=True)
    i1 = jnp.min(jnp.where(biased == m1, eidx, big), axis=-1, keepdims=True)
    masked = jnp.where(eidx == i1, -jnp.inf, biased)
    m2 = jnp.max(masked, axis=-1, keepdims=True)
    i2 = jnp.min(jnp.where(masked == m2, eidx, big), axis=-1, keepdims=True)

    denom = m1 + m2
    is1 = (eidx == i1)
    is2 = (eidx == i2)
    coeff = jnp.where(is1, m1 / denom, 0.0) + jnp.where(is2, m2 / denom, 0.0)

    cnt = jnp.sum(is1.astype(jnp.float32) + is2.astype(jnp.float32),
                  axis=0, keepdims=True)
    importance = jnp.mean(route_prob, axis=0, keepdims=True)
    load = cnt / jnp.maximum(jnp.sum(cnt), 1.0)
    lb = jnp.float32(E) * jnp.sum(importance * load)

    g = jnp.dot(coeff, gamma, preferred_element_type=jnp.float32)
    bvec = jnp.dot(coeff, beta, preferred_element_type=jnp.float32)
    return coeff, lb, g, bvec


def _norm_block(xb, g, bvec):
    s1 = jnp.mean(xb, axis=-1, keepdims=True)
    s2 = jnp.mean(xb * xb, axis=-1, keepdims=True)
    rstd = jax.lax.rsqrt(s2 - s1 * s1 + 1e-6)
    return (xb - s1) * rstd * g + bvec


def _mega_kernel(x_ref, w1_ref, b1_ref, w2_ref, b2_ref, pen_ref,
                 gamma_ref, beta_ref,
                 o_ref, coeff_ref, lb_ref,
                 acc_pool, acc_h, gb_scr, res):
    t = pl.program_id(0)

    @pl.when(t == 0)
    def _():
        acc_pool[...] = jnp.zeros_like(acc_pool)

    @pl.when(t < T_POOL)
    def _():
        bidx = t // CPS
        part = jnp.zeros((D,), jnp.float32)
        for i in range(S_CHUNK // 128):
            part += jnp.sum(x_ref[0, i * 128:(i + 1) * 128], axis=0)
        acc_pool[bidx, :] += part * (1.0 / S)

        @pl.when(t < N_RES)
        def _():
            for i in range(S_CHUNK // 128):
                res[t, i * 128:(i + 1) * 128] = (
                    x_ref[0, i * 128:(i + 1) * 128].astype(jnp.bfloat16))

    @pl.when(t == T_POOL)
    def _():
        acc_h[...] = jnp.zeros_like(acc_h)

    @pl.when((t >= T_POOL) & (t < T_MM))
    def _():
        j = t - T_POOL
        acc_h[...] += jnp.dot(acc_pool[:, pl.ds(j * D_CHUNK, D_CHUNK)],
                              w1_ref[...], preferred_element_type=jnp.float32)

    @pl.when(t == T_MM - 1)
    def _():
        h = jnp.maximum(acc_h[...] + b1_ref[...][None, :], 0.0)
        coeff, lb, g, bvec = _routing_tail(
            h, w2_ref[...], b2_ref[...], pen_ref[...],
            gamma_ref[...], beta_ref[...])
        coeff_ref[...] = coeff
        lb_ref[...] = lb.reshape(1, 1)
        gb_scr[0] = g
        gb_scr[1] = bvec

    @pl.when(t >= T_MM)
    def _():
        c = t - T_MM
        bidx = c // CPS
        g = gb_scr[0, bidx][None, :]
        bvec = gb_scr[1, bidx][None, :]

        @pl.when(c < N_RES)
        def _():
            for i in range(S_CHUNK // 128):
                sl = slice(i * 128, (i + 1) * 128)
                o_ref[0, sl] = _norm_block(
                    res[c, sl].astype(jnp.float32), g, bvec)

        @pl.when(c >= N_RES)
        def _():
            for i in range(S_CHUNK // 128):
                sl = slice(i * 128, (i + 1) * 128)
                o_ref[0, sl] = _norm_block(x_ref[0, sl], g, bvec)


def _x_idx(t):
    c = jnp.where(t < T_POOL, t, jnp.where(t < T_MM + N_RES, NC - 1, t - T_MM))
    return (c // CPS, c % CPS, 0)


def _o_idx(t):
    c = jnp.where(t < T_MM, 0, t - T_MM)
    return (c // CPS, c % CPS, 0)


def _w1_idx(t):
    j = jnp.clip(t - T_POOL, 0, NJ - 1)
    return (j, 0)


@jax.jit
def kernel(x, router_w1, router_b1, router_w2, router_b2, gamma, beta, penalty):
    out, coeff, lb = pl.pallas_call(
        _mega_kernel,
        grid=(T_END,),
        in_specs=[
            pl.BlockSpec((1, S_CHUNK, D), _x_idx),
            pl.BlockSpec((D_CHUNK, D), _w1_idx),
            pl.BlockSpec((D,), lambda t: (0,)),
            pl.BlockSpec((E, D), lambda t: (0, 0)),
            pl.BlockSpec((E,), lambda t: (0,)),
            pl.BlockSpec((E,), lambda t: (0,)),
            pl.BlockSpec((E, D), lambda t: (0, 0)),
            pl.BlockSpec((E, D), lambda t: (0, 0)),
        ],
        out_specs=[
            pl.BlockSpec((1, S_CHUNK, D), _o_idx),
            pl.BlockSpec((B, E), lambda t: (0, 0)),
            pl.BlockSpec((1, 1), lambda t: (0, 0)),
        ],
        out_shape=[
            jax.ShapeDtypeStruct((B, S, D), jnp.float32),
            jax.ShapeDtypeStruct((B, E), jnp.float32),
            jax.ShapeDtypeStruct((1, 1), jnp.float32),
        ],
        scratch_shapes=[
            pltpu.VMEM((B, D), jnp.float32),
            pltpu.VMEM((B, D), jnp.float32),
            pltpu.VMEM((2, B, D), jnp.float32),
            pltpu.VMEM((N_RES, S_CHUNK, D), jnp.bfloat16),
        ],
        compiler_params=pltpu.CompilerParams(
            dimension_semantics=("arbitrary",),
            vmem_limit_bytes=67108864),
    )(x, router_w1, router_b1, router_w2.T, router_b2, penalty, gamma, beta)

    return (out, coeff, lb.reshape(()))
